# baseline (speedup 1.0000x reference)
import jax
import jax.numpy as jnp
from jax.experimental import pallas as pl
from jax.experimental.pallas import tpu as pltpu

_BATCH = 8
_HID = 128


def _mlp_kernel(idx_ref,
                x_ref, W_in_ref, b_in_ref,
                g0W_ref, g0b_ref, g1W_ref, g1b_ref, g2W_ref, g2b_ref,
                emb_ref, Wf_ref, bf_ref,
                Wp1_ref, bp1_ref, Wp2_ref, bp2_ref, Wp3_ref, bp3_ref,
                Wd1_ref, bd1_ref, Wd2_ref, bd2_ref, Wd3_ref, bd3_ref,
                out_ref, emb_scratch, sems):
    f32 = jnp.float32

    def mm(a, w):
        return jax.lax.dot_general(a, w, (((1,), (0,)), ((), ())),
                                   preferred_element_type=f32)

    copies = [pltpu.make_async_copy(emb_ref.at[pl.ds(idx_ref[i], 1), :],
                                    emb_scratch.at[pl.ds(i, 1), :],
                                    sems.at[i])
              for i in range(_BATCH)]
    for c in copies:
        c.start()

    v = jnp.maximum(mm(x_ref[...], W_in_ref[...]) + b_in_ref[...], 0.0)
    v = jnp.maximum(mm(v, g0W_ref[...]) + g0b_ref[...], 0.0)
    v = jnp.maximum(mm(v, g1W_ref[...]) + g1b_ref[...], 0.0)
    v = jnp.maximum(mm(v, g2W_ref[...]) + g2b_ref[...], 0.0)

    for c in copies:
        c.wait()
    emb = emb_scratch[...]

    wf = Wf_ref[...]
    fused = jnp.maximum(mm(v, wf[0:_HID, :]) + mm(emb, wf[_HID:_HID + 32, :])
                        + bf_ref[...], 0.0)

    h = jnp.maximum(mm(fused, Wp1_ref[...]) + bp1_ref[...], 0.0)
    h = jnp.maximum(mm(h, Wp2_ref[...]) + bp2_ref[...], 0.0)
    price = mm(h, Wp3_ref[...]) + bp3_ref[...]

    h2 = jnp.maximum(mm(fused, Wd1_ref[...]) + bd1_ref[...], 0.0)
    h2 = jnp.maximum(mm(h2, Wd2_ref[...]) + bd2_ref[...], 0.0)
    direction = jax.nn.sigmoid(mm(h2, Wd3_ref[...]) + bd3_ref[...])

    out_ref[...] = jnp.concatenate([price, direction], axis=1)


def kernel(x, company_indices, edge_index, edge_attr,
           W_in, b_in,
           gat0_W, gat0_att_src, gat0_att_dst, gat0_We, gat0_att_edge, gat0_b,
           gat1_W, gat1_att_src, gat1_att_dst, gat1_We, gat1_att_edge, gat1_b,
           gat2_W, gat2_att_src, gat2_att_dst, gat2_We, gat2_att_edge, gat2_b,
           emb_table, W_fuse, b_fuse,
           Wp1, bp1, Wp2, bp2, Wp3, bp3,
           Wd1, bd1, Wd2, bd2, Wd3, bd3):
    idx = company_indices.astype(jnp.int32)

    row = lambda b: b.reshape(1, -1)
    args = (
        x, W_in, row(b_in),
        gat0_W, row(gat0_b), gat1_W, row(gat1_b), gat2_W, row(gat2_b),
        emb_table, W_fuse, row(b_fuse),
        Wp1, row(bp1), Wp2, row(bp2), Wp3, bp3.reshape(1, 1),
        Wd1, row(bd1), Wd2, row(bd2), Wd3, bd3.reshape(1, 1),
    )

    in_specs = [pl.BlockSpec(memory_space=pltpu.SMEM)]
    for a in args:
        if a is emb_table:
            in_specs.append(pl.BlockSpec(memory_space=pltpu.MemorySpace.HBM))
        else:
            in_specs.append(pl.BlockSpec(a.shape, lambda *_: (0,) * a.ndim))

    out = pl.pallas_call(
        _mlp_kernel,
        out_shape=jax.ShapeDtypeStruct((_BATCH, 2), jnp.float32),
        in_specs=in_specs,
        out_specs=pl.BlockSpec((_BATCH, 2), lambda *_: (0, 0)),
        scratch_shapes=[pltpu.VMEM((_BATCH, emb_table.shape[1]), jnp.float32),
                        pltpu.SemaphoreType.DMA((_BATCH,))],
    )(idx, *args)

    return out[:, 0], out[:, 1]


# speedup vs baseline: 1.0889x; 1.0889x over previous
import jax
import jax.numpy as jnp
from jax.experimental import pallas as pl
from jax.experimental.pallas import tpu as pltpu

_BATCH = 8
_HID = 128


def _mlp_kernel(idx_ref,
                x_ref, W_in_ref, b_in_ref,
                g0W_ref, g0b_ref, g1W_ref, g1b_ref, g2W_ref, g2b_ref,
                emb_ref, Wf_ref, bf_ref,
                Wp1_ref, bp1_ref, Wp2_ref, bp2_ref, Wp3_ref, bp3_ref,
                Wd1_ref, bd1_ref, Wd2_ref, bd2_ref, Wd3_ref, bd3_ref,
                price_ref, dir_ref, emb_scratch, sems):
    f32 = jnp.float32

    def mm(a, w):
        return jax.lax.dot_general(a, w, (((1,), (0,)), ((), ())),
                                   preferred_element_type=f32)

    copies = [pltpu.make_async_copy(emb_ref.at[pl.ds(idx_ref[i], 1), :],
                                    emb_scratch.at[pl.ds(i, 1), :],
                                    sems.at[i])
              for i in range(_BATCH)]
    for c in copies:
        c.start()

    v = jnp.maximum(mm(x_ref[...], W_in_ref[...]) + b_in_ref[...], 0.0)
    v = jnp.maximum(mm(v, g0W_ref[...]) + g0b_ref[...], 0.0)
    v = jnp.maximum(mm(v, g1W_ref[...]) + g1b_ref[...], 0.0)
    v = jnp.maximum(mm(v, g2W_ref[...]) + g2b_ref[...], 0.0)

    for c in copies:
        c.wait()
    emb = emb_scratch[...]

    wf = Wf_ref[...]
    fused = jnp.maximum(mm(v, wf[0:_HID, :]) + mm(emb, wf[_HID:_HID + 32, :])
                        + bf_ref[...], 0.0)

    h = jnp.maximum(mm(fused, Wp1_ref[...]) + bp1_ref[...], 0.0)
    h = jnp.maximum(mm(h, Wp2_ref[...]) + bp2_ref[...], 0.0)
    price = mm(h, Wp3_ref[...]) + bp3_ref[...]

    h2 = jnp.maximum(mm(fused, Wd1_ref[...]) + bd1_ref[...], 0.0)
    h2 = jnp.maximum(mm(h2, Wd2_ref[...]) + bd2_ref[...], 0.0)
    direction = jax.nn.sigmoid(mm(h2, Wd3_ref[...]) + bd3_ref[...])

    price_ref[...] = price.reshape(_BATCH)
    dir_ref[...] = direction.reshape(_BATCH)


def kernel(x, company_indices, edge_index, edge_attr,
           W_in, b_in,
           gat0_W, gat0_att_src, gat0_att_dst, gat0_We, gat0_att_edge, gat0_b,
           gat1_W, gat1_att_src, gat1_att_dst, gat1_We, gat1_att_edge, gat1_b,
           gat2_W, gat2_att_src, gat2_att_dst, gat2_We, gat2_att_edge, gat2_b,
           emb_table, W_fuse, b_fuse,
           Wp1, bp1, Wp2, bp2, Wp3, bp3,
           Wd1, bd1, Wd2, bd2, Wd3, bd3):
    idx = company_indices.astype(jnp.int32)

    row = lambda b: b.reshape(1, -1)
    args = (
        x, W_in, row(b_in),
        gat0_W, row(gat0_b), gat1_W, row(gat1_b), gat2_W, row(gat2_b),
        emb_table, W_fuse, row(b_fuse),
        Wp1, row(bp1), Wp2, row(bp2), Wp3, bp3.reshape(1, 1),
        Wd1, row(bd1), Wd2, row(bd2), Wd3, bd3.reshape(1, 1),
    )

    in_specs = [pl.BlockSpec(memory_space=pltpu.SMEM)]
    for a in args:
        if a is emb_table:
            in_specs.append(pl.BlockSpec(memory_space=pltpu.MemorySpace.HBM))
        else:
            in_specs.append(pl.BlockSpec(a.shape, lambda *_: (0,) * a.ndim))

    price, direction = pl.pallas_call(
        _mlp_kernel,
        out_shape=[jax.ShapeDtypeStruct((_BATCH,), jnp.float32),
                   jax.ShapeDtypeStruct((_BATCH,), jnp.float32)],
        in_specs=in_specs,
        out_specs=[pl.BlockSpec((_BATCH,), lambda *_: (0,)),
                   pl.BlockSpec((_BATCH,), lambda *_: (0,))],
        scratch_shapes=[pltpu.VMEM((_BATCH, emb_table.shape[1]), jnp.float32),
                        pltpu.SemaphoreType.DMA((_BATCH,))],
    )(idx, *args)

    return price, direction
